# Initial kernel scaffold; baseline (speedup 1.0000x reference)
import jax
import jax.numpy as jnp
from jax.experimental import pallas as pl


def kernel(x, edge_index, emb, W_lin, b_lin, att_i, att_j, att_em_i, att_em_j, gnn_bias, bn1_gamma, bn1_beta, bn2_gamma, bn2_beta, W_out, b_out):
    raise NotImplementedError("write your pallas kernel here")


# speedup vs baseline: 4.2460x; 4.2460x over previous
import jax
import jax.numpy as jnp
from jax.experimental import pallas as pl
from jax.experimental.pallas import tpu as pltpu

_N = 10000
_D = 64
_K = 20
_R = 200


def _split2(a):
    a1 = a.astype(jnp.bfloat16)
    a2 = (a - a1.astype(jnp.float32)).astype(jnp.bfloat16)
    return a1, a2


def _split3(a):
    a1 = a.astype(jnp.bfloat16)
    r = a - a1.astype(jnp.float32)
    a2 = r.astype(jnp.bfloat16)
    a3 = (r - a2.astype(jnp.float32)).astype(jnp.bfloat16)
    return a1, a2, a3


def _mm(a, b):
    return jnp.dot(a, b, preferred_element_type=jnp.float32)


def _dot_x6(a, b):
    a1, a2, a3 = _split3(a)
    b1, b2, b3 = _split3(b)
    lo = _mm(a3, b1) + _mm(a2, b2) + _mm(a1, b3)
    mid = _mm(a2, b1) + _mm(a1, b2)
    return (lo + mid) + _mm(a1, b1)


def _dot_x3(a, b):
    a1, a2 = _split2(a)
    b1, b2 = _split2(b)
    return (_mm(a2, b1) + _mm(a1, b2)) + _mm(a1, b1)


def _pre_kernel(x_ref, wl_ref, bl_ref, emb_ref, ati_ref, atj_ref, atei_ref,
                atej_ref, h_ref, inv_ref, ai_ref, aj_ref):
    h = _mm(x_ref[:].astype(jnp.bfloat16),
            wl_ref[:].astype(jnp.bfloat16)) + bl_ref[:]
    h_ref[:] = h
    e = emb_ref[:]
    nrm2 = jnp.maximum(jnp.sum(e * e, axis=1, keepdims=True), 1e-24)
    r = jax.lax.rsqrt(nrm2)
    r = r * (1.5 - 0.5 * nrm2 * r * r)
    inv_ref[:] = jnp.minimum(r, 1e12)
    ai_ref[:] = (jnp.sum(h * ati_ref[:], axis=1, keepdims=True)
                 + jnp.sum(e * atei_ref[:], axis=1, keepdims=True))
    aj_ref[:] = (jnp.sum(h * atj_ref[:], axis=1, keepdims=True)
                 + jnp.sum(e * atej_ref[:], axis=1, keepdims=True))


def _attn_kernel(emb_t_ref, embT_ref, ai_t_ref, ajr_ref, h_ref, gb_ref,
                 invr_ref, o_ref):
    s = _mm(emb_t_ref[:], embT_ref[:]) * invr_ref[:]
    iota = jax.lax.broadcasted_iota(jnp.int32, (_R, _N), 1)
    sel = jnp.zeros((_R, _N), jnp.bool_)
    for _ in range(_K):
        m = jnp.max(s, axis=1, keepdims=True)
        key = jnp.where(s == m, iota, jnp.int32(2**30))
        p = jnp.min(key, axis=1, keepdims=True)
        hit = iota == p
        sel = jnp.logical_or(sel, hit)
        s = jnp.where(hit, -jnp.inf, s)
    alpha = ai_t_ref[:] + ajr_ref[:]
    alpha = jnp.where(alpha >= 0, alpha, 0.2 * alpha)
    af = jnp.where(sel, alpha, -jnp.inf)
    m2 = jnp.max(af, axis=1, keepdims=True)
    ex = jnp.exp(af - m2)
    ssum = jnp.sum(ex, axis=1, keepdims=True)
    att = ex * (1.0 / (ssum + 1e-16))
    o_ref[:] = _dot_x3(att, h_ref[:]) + gb_ref[:]


def _post_kernel(o_ref, emb_ref, g1_ref, b1_ref, g2_ref, b2_ref, wo_ref,
                 bo_ref, z_ref):
    o = o_ref[:]
    mu = jnp.mean(o, axis=0, keepdims=True)
    var = jnp.mean((o - mu) * (o - mu), axis=0, keepdims=True)
    o = (o - mu) * jax.lax.rsqrt(var + 1e-5) * g1_ref[:] + b1_ref[:]
    o = jnp.maximum(o, 0.0)
    y = o * emb_ref[:]
    mu2 = jnp.mean(y, axis=0, keepdims=True)
    var2 = jnp.mean((y - mu2) * (y - mu2), axis=0, keepdims=True)
    y = (y - mu2) * jax.lax.rsqrt(var2 + 1e-5) * g2_ref[:] + b2_ref[:]
    y = jnp.maximum(y, 0.0)
    z = _mm(y.astype(jnp.bfloat16),
            wo_ref[:].astype(jnp.bfloat16)) + bo_ref[:]
    z_ref[:] = jnp.tanh(z)


def kernel(x, edge_index, emb, W_lin, b_lin, att_i, att_j, att_em_i, att_em_j,
           gnn_bias, bn1_gamma, bn1_beta, bn2_gamma, bn2_beta, W_out, b_out):
    del edge_index
    b, n, f = x.shape
    p_out = W_out.shape[1]
    xf = x.reshape(n, f)

    rp = 1000 if n % 1000 == 0 else n
    h, inv, ai, aj = pl.pallas_call(
        _pre_kernel,
        grid=(n // rp,),
        in_specs=[
            pl.BlockSpec((rp, f), lambda i: (i, 0)),
            pl.BlockSpec((f, _D), lambda i: (0, 0)),
            pl.BlockSpec((1, _D), lambda i: (0, 0)),
            pl.BlockSpec((rp, _D), lambda i: (i, 0)),
            pl.BlockSpec((1, _D), lambda i: (0, 0)),
            pl.BlockSpec((1, _D), lambda i: (0, 0)),
            pl.BlockSpec((1, _D), lambda i: (0, 0)),
            pl.BlockSpec((1, _D), lambda i: (0, 0)),
        ],
        out_specs=[
            pl.BlockSpec((rp, _D), lambda i: (i, 0)),
            pl.BlockSpec((rp, 1), lambda i: (i, 0)),
            pl.BlockSpec((rp, 1), lambda i: (i, 0)),
            pl.BlockSpec((rp, 1), lambda i: (i, 0)),
        ],
        out_shape=[
            jax.ShapeDtypeStruct((n, _D), jnp.float32),
            jax.ShapeDtypeStruct((n, 1), jnp.float32),
            jax.ShapeDtypeStruct((n, 1), jnp.float32),
            jax.ShapeDtypeStruct((n, 1), jnp.float32),
        ],
        compiler_params=pltpu.CompilerParams(
            dimension_semantics=("parallel",)),
    )(xf, W_lin, b_lin.reshape(1, _D), emb, att_i.reshape(1, _D),
      att_j.reshape(1, _D), att_em_i.reshape(1, _D), att_em_j.reshape(1, _D))

    emb_bf = emb.astype(jnp.bfloat16)
    ajr = aj.reshape(1, n)
    invr = inv.reshape(1, n)

    out = pl.pallas_call(
        _attn_kernel,
        grid=(n // _R,),
        in_specs=[
            pl.BlockSpec((_R, _D), lambda i: (i, 0)),
            pl.BlockSpec((_D, n), lambda i: (0, 0)),
            pl.BlockSpec((_R, 1), lambda i: (i, 0)),
            pl.BlockSpec((1, n), lambda i: (0, 0)),
            pl.BlockSpec((n, _D), lambda i: (0, 0)),
            pl.BlockSpec((1, _D), lambda i: (0, 0)),
            pl.BlockSpec((1, n), lambda i: (0, 0)),
        ],
        out_specs=pl.BlockSpec((_R, _D), lambda i: (i, 0)),
        out_shape=jax.ShapeDtypeStruct((n, _D), jnp.float32),
        compiler_params=pltpu.CompilerParams(
            dimension_semantics=("parallel",)),
    )(emb_bf, emb_bf.T, ai, ajr, h, gnn_bias.reshape(1, _D), invr)

    z = pl.pallas_call(
        _post_kernel,
        out_shape=jax.ShapeDtypeStruct((n, p_out), jnp.float32),
    )(out, emb, bn1_gamma.reshape(1, _D), bn1_beta.reshape(1, _D),
      bn2_gamma.reshape(1, _D), bn2_beta.reshape(1, _D), W_out,
      b_out.reshape(1, p_out))

    return z.reshape(b, n, p_out)
